# Initial kernel scaffold; baseline (speedup 1.0000x reference)
import jax
import jax.numpy as jnp
from jax.experimental import pallas as pl


def kernel(state, W1, b1, W2, b2, We, be, Wv, bv):
    raise NotImplementedError("write your pallas kernel here")


# speedup vs baseline: 1.0518x; 1.0518x over previous
import jax
import jax.numpy as jnp
from jax.experimental import pallas as pl
from jax.experimental.pallas import tpu as pltpu

_BLK = 512


def _fused_body(x_ref, w1_ref, b1_ref, w2_ref, b2_ref, we_ref, be_ref,
                wv_ref, bv_ref, ew_ref, val_ref):
    nt = (((1,), (1,)), ((), ()))
    x = x_ref[...].astype(jnp.bfloat16)
    h = jax.lax.dot_general(x, w1_ref[...], nt,
                            preferred_element_type=jnp.float32)
    h = jnp.maximum(h + b1_ref[...], 0.0).astype(jnp.bfloat16)
    h = jax.lax.dot_general(h, w2_ref[...], nt,
                            preferred_element_type=jnp.float32)
    h = jnp.maximum(h + b2_ref[...], 0.0).astype(jnp.bfloat16)
    logits = jax.lax.dot_general(h, we_ref[...], nt,
                                 preferred_element_type=jnp.float32)
    logits = logits + be_ref[...]
    m = jnp.max(logits, axis=-1, keepdims=True)
    e = jnp.exp(logits - m)
    ew_ref[...] = e / jnp.sum(e, axis=-1, keepdims=True)
    val = jnp.sum(h.astype(jnp.float32) * wv_ref[...], axis=-1, keepdims=True)
    val_ref[...] = val + bv_ref[...]


def kernel(state, W1, b1, W2, b2, We, be, Wv, bv):
    B, D = state.shape
    H = W1.shape[0]
    E = We.shape[0]
    w1 = W1.astype(jnp.bfloat16)
    w2 = W2.astype(jnp.bfloat16)
    we = We.astype(jnp.bfloat16)
    wv = Wv.reshape(1, H)
    b1r = b1.reshape(1, H)
    b2r = b2.reshape(1, H)
    ber = be.reshape(1, E)
    bvr = bv.reshape(1, 1)

    grid = (B // _BLK,)
    row = lambda i: (i, 0)
    full = lambda i: (0, 0)
    ew, val = pl.pallas_call(
        _fused_body,
        grid=grid,
        in_specs=[
            pl.BlockSpec((_BLK, D), row),
            pl.BlockSpec((H, D), full),
            pl.BlockSpec((1, H), full),
            pl.BlockSpec((H, H), full),
            pl.BlockSpec((1, H), full),
            pl.BlockSpec((E, H), full),
            pl.BlockSpec((1, E), full),
            pl.BlockSpec((1, H), full),
            pl.BlockSpec((1, 1), full),
        ],
        out_specs=[
            pl.BlockSpec((_BLK, E), row),
            pl.BlockSpec((_BLK, 1), row),
        ],
        out_shape=[
            jax.ShapeDtypeStruct((B, E), jnp.float32),
            jax.ShapeDtypeStruct((B, 1), jnp.float32),
        ],
    )(state, w1, b1r, w2, b2r, we, ber, wv, bvr)
    return ew, val


# speedup vs baseline: 1.0682x; 1.0155x over previous
import jax
import jax.numpy as jnp
from jax.experimental import pallas as pl
from jax.experimental.pallas import tpu as pltpu

_BLK = 1024


def _fused_body(x_ref, w1_ref, b1_ref, w2_ref, b2_ref, we_ref, be_ref,
                wv_ref, bv_ref, ew_ref, val_ref):
    nt = (((1,), (1,)), ((), ()))
    x = x_ref[...].astype(jnp.bfloat16)
    h = jax.lax.dot_general(x, w1_ref[...], nt,
                            preferred_element_type=jnp.float32)
    h = jnp.maximum(h + b1_ref[...], 0.0).astype(jnp.bfloat16)
    h = jax.lax.dot_general(h, w2_ref[...], nt,
                            preferred_element_type=jnp.float32)
    h = jnp.maximum(h + b2_ref[...], 0.0).astype(jnp.bfloat16)
    logits = jax.lax.dot_general(h, we_ref[...], nt,
                                 preferred_element_type=jnp.float32)
    logits = logits + be_ref[...]
    m = jnp.max(logits, axis=-1, keepdims=True)
    e = jnp.exp(logits - m)
    ew_ref[...] = e / jnp.sum(e, axis=-1, keepdims=True)
    val = jnp.sum(h.astype(jnp.float32) * wv_ref[...], axis=-1, keepdims=True)
    val_ref[...] = val + bv_ref[...]


def kernel(state, W1, b1, W2, b2, We, be, Wv, bv):
    B, D = state.shape
    H = W1.shape[0]
    E = We.shape[0]
    w1 = W1.astype(jnp.bfloat16)
    w2 = W2.astype(jnp.bfloat16)
    we = We.astype(jnp.bfloat16)
    wv = Wv.reshape(1, H)
    b1r = b1.reshape(1, H)
    b2r = b2.reshape(1, H)
    ber = be.reshape(1, E)
    bvr = bv.reshape(1, 1)

    grid = (B // _BLK,)
    row = lambda i: (i, 0)
    full = lambda i: (0, 0)
    ew, val = pl.pallas_call(
        _fused_body,
        grid=grid,
        in_specs=[
            pl.BlockSpec((_BLK, D), row),
            pl.BlockSpec((H, D), full),
            pl.BlockSpec((1, H), full),
            pl.BlockSpec((H, H), full),
            pl.BlockSpec((1, H), full),
            pl.BlockSpec((E, H), full),
            pl.BlockSpec((1, E), full),
            pl.BlockSpec((1, H), full),
            pl.BlockSpec((1, 1), full),
        ],
        out_specs=[
            pl.BlockSpec((_BLK, E), row),
            pl.BlockSpec((_BLK, 1), row),
        ],
        out_shape=[
            jax.ShapeDtypeStruct((B, E), jnp.float32),
            jax.ShapeDtypeStruct((B, 1), jnp.float32),
        ],
        compiler_params=pltpu.CompilerParams(
            dimension_semantics=("parallel",),
        ),
    )(state, w1, b1r, w2, b2r, we, ber, wv, bvr)
    return ew, val


# speedup vs baseline: 1.1487x; 1.0754x over previous
import jax
import jax.numpy as jnp
from jax.experimental import pallas as pl
from jax.experimental.pallas import tpu as pltpu

_BLK = 512


def _fused_body(x_ref, w1_ref, b1_ref, w2_ref, b2_ref, we_ref, be_ref,
                wv_ref, bv_ref, ew_ref, val_ref, w1b, w2b, web):
    @pl.when(pl.program_id(0) == 0)
    def _convert_weights():
        w1b[...] = w1_ref[...].astype(jnp.bfloat16)
        w2b[...] = w2_ref[...].astype(jnp.bfloat16)
        web[...] = we_ref[...].astype(jnp.bfloat16)

    nt = (((1,), (1,)), ((), ()))
    x = x_ref[...].astype(jnp.bfloat16)
    h = jax.lax.dot_general(x, w1b[...], nt,
                            preferred_element_type=jnp.float32)
    h = jnp.maximum(h + b1_ref[...], 0.0).astype(jnp.bfloat16)
    h = jax.lax.dot_general(h, w2b[...], nt,
                            preferred_element_type=jnp.float32)
    h = jnp.maximum(h + b2_ref[...], 0.0).astype(jnp.bfloat16)
    logits = jax.lax.dot_general(h, web[...], nt,
                                 preferred_element_type=jnp.float32)
    logits = logits + be_ref[...]
    m = jnp.max(logits, axis=-1, keepdims=True)
    e = jnp.exp(logits - m)
    ew_ref[...] = e / jnp.sum(e, axis=-1, keepdims=True)
    val = jnp.sum(h.astype(jnp.float32) * wv_ref[...], axis=-1, keepdims=True)
    val_ref[...] = val + bv_ref[...]


def kernel(state, W1, b1, W2, b2, We, be, Wv, bv):
    B, D = state.shape
    H = W1.shape[0]
    E = We.shape[0]
    b1r = b1.reshape(1, H)
    b2r = b2.reshape(1, H)
    ber = be.reshape(1, E)
    bvr = bv.reshape(1, 1)
    wvr = Wv.reshape(1, H)

    grid = (B // _BLK,)
    row = lambda i: (i, 0)
    full = lambda i: (0, 0)
    ew, val = pl.pallas_call(
        _fused_body,
        grid=grid,
        in_specs=[
            pl.BlockSpec((_BLK, D), row),
            pl.BlockSpec((H, D), full),
            pl.BlockSpec((1, H), full),
            pl.BlockSpec((H, H), full),
            pl.BlockSpec((1, H), full),
            pl.BlockSpec((E, H), full),
            pl.BlockSpec((1, E), full),
            pl.BlockSpec((1, H), full),
            pl.BlockSpec((1, 1), full),
        ],
        out_specs=[
            pl.BlockSpec((_BLK, E), row),
            pl.BlockSpec((_BLK, 1), row),
        ],
        out_shape=[
            jax.ShapeDtypeStruct((B, E), jnp.float32),
            jax.ShapeDtypeStruct((B, 1), jnp.float32),
        ],
        scratch_shapes=[
            pltpu.VMEM((H, D), jnp.bfloat16),
            pltpu.VMEM((H, H), jnp.bfloat16),
            pltpu.VMEM((E, H), jnp.bfloat16),
        ],
        compiler_params=pltpu.CompilerParams(
            dimension_semantics=("arbitrary",),
        ),
    )(state, W1, b1r, W2, b2r, We, ber, wvr, bvr)
    return ew, val
